# baseline (speedup 1.0000x reference)
import functools

import jax
import jax.numpy as jnp
from jax import lax
from jax.experimental import pallas as pl
from jax.experimental.pallas import tpu as pltpu
from jax.experimental.pallas import tpu_sc as plsc

D = 64
L = 200
W = 40
NG = L // W
NC, NS = 2, 16
NW = NC * NS
UNROLL = 8


def _sc_pool(ids2, table, batch):
    bpw = batch // NW
    mesh = plsc.VectorSubcoreMesh(core_axis_name="c", subcore_axis_name="s")

    @functools.partial(
        pl.kernel,
        mesh=mesh,
        out_type=jax.ShapeDtypeStruct((batch, D), jnp.float32),
        compiler_params=pltpu.CompilerParams(use_tc_tiling_on_sc=False),
        scratch_types=[
            pltpu.VMEM((NG * bpw, W), jnp.int32),
            pltpu.VMEM((L, 2 * D), jnp.float32),
            pltpu.VMEM((L, 2 * D), jnp.float32),
            pltpu.VMEM((bpw, D), jnp.float32),
            pltpu.SemaphoreType.DMA,
            pltpu.SemaphoreType.DMA,
        ],
    )
    def pool_kernel(ids_hbm, table_hbm, out_hbm, ids_v, rows_a, rows_b,
                    pool_v, sem_a, sem_b):
        wid = lax.axis_index("s") * NC + lax.axis_index("c")
        base = wid * bpw

        pltpu.sync_copy(ids_hbm.at[pl.ds(NG * base, NG * bpw)], ids_v)

        def issue(s, rows, sem):
            for c in range(NG):
                pltpu.async_copy(table_hbm.at[ids_v.at[NG * s + c]],
                                 rows.at[pl.ds(W * c, W)], sem)

        def wait(rows, sem):
            pltpu.make_async_copy(table_hbm.at[pl.ds(0, L)], rows, sem).wait()

        def accum(rows, s):
            def body(k, acc):
                i = UNROLL * k
                for u in range(UNROLL):
                    acc = tuple(
                        acc[j] + rows[i + u, pl.ds(16 * j, 16)]
                        for j in range(4)
                    )
                return acc

            zero = jnp.zeros((16,), jnp.float32)
            acc = lax.fori_loop(0, L // UNROLL, body, (zero,) * 4)
            scale = jnp.float32(1.0 / L)
            for j in range(4):
                pool_v[s, pl.ds(16 * j, 16)] = acc[j] * scale

        issue(0, rows_a, sem_a)

        def outer(k, carry):
            s = 2 * k
            issue(s + 1, rows_b, sem_b)
            wait(rows_a, sem_a)
            accum(rows_a, s)

            @pl.when(k < bpw // 2 - 1)
            def _():
                issue(s + 2, rows_a, sem_a)

            wait(rows_b, sem_b)
            accum(rows_b, s + 1)
            return carry

        lax.fori_loop(0, bpw // 2, outer, 0)

        pltpu.sync_copy(pool_v, out_hbm.at[pl.ds(base, bpw)])

    return pool_kernel(ids2, table)


def _mlp_body(x_ref, w1_ref, b1_ref, w2_ref, b2_ref, o_ref):
    h = jnp.dot(x_ref[...], w1_ref[...],
                preferred_element_type=jnp.float32) + b1_ref[...]
    h = jnp.maximum(h, 0.0)
    o_ref[...] = jnp.dot(h, w2_ref[...],
                         preferred_element_type=jnp.float32) + b2_ref[...]


@jax.jit
def kernel(text_ids, table, W1, b1, W2, b2):
    batch = text_ids.shape[0]
    ids2 = text_ids.astype(jnp.int32).reshape(NG * batch, W)
    table2 = jnp.pad(table, ((0, 0), (0, D)))
    pooled = _sc_pool(ids2, table2, batch)
    return pl.pallas_call(
        _mlp_body,
        out_shape=jax.ShapeDtypeStruct((batch, D), jnp.float32),
    )(pooled, W1, b1.reshape(1, D), W2, b2.reshape(1, D))


# speedup vs baseline: 1.0256x; 1.0256x over previous
import functools

import jax
import jax.numpy as jnp
from jax import lax
from jax.experimental import pallas as pl
from jax.experimental.pallas import tpu as pltpu
from jax.experimental.pallas import tpu_sc as plsc

D = 64
L = 200
W = 40
NG = L // W
NC, NS = 2, 16
NW = NC * NS
UNROLL = 8


def _sc_pool(ids2, table, batch):
    bpw = batch // NW
    mesh = plsc.VectorSubcoreMesh(core_axis_name="c", subcore_axis_name="s")

    @functools.partial(
        pl.kernel,
        mesh=mesh,
        out_type=jax.ShapeDtypeStruct((batch, D), jnp.float32),
        compiler_params=pltpu.CompilerParams(use_tc_tiling_on_sc=False),
        scratch_types=[
            pltpu.VMEM((NG * bpw, W), jnp.int32),
            pltpu.VMEM((L, D), jnp.float32),
            pltpu.VMEM((L, D), jnp.float32),
            pltpu.VMEM((bpw, D), jnp.float32),
            pltpu.SemaphoreType.DMA,
            pltpu.SemaphoreType.DMA,
        ],
    )
    def pool_kernel(ids_hbm, table_hbm, out_hbm, ids_v, rows_a, rows_b,
                    pool_v, sem_a, sem_b):
        wid = lax.axis_index("s") * NC + lax.axis_index("c")
        base = wid * bpw

        pltpu.sync_copy(ids_hbm.at[pl.ds(NG * base, NG * bpw)], ids_v)

        def issue(s, rows, sem):
            for c in range(NG):
                pltpu.async_copy(table_hbm.at[ids_v.at[NG * s + c]],
                                 rows.at[pl.ds(W * c, W)], sem)

        def wait(rows, sem):
            pltpu.make_async_copy(table_hbm.at[pl.ds(0, L)], rows, sem).wait()

        def accum(rows, s):
            def body(k, acc):
                i = UNROLL * k
                for u in range(UNROLL):
                    acc = tuple(
                        acc[j] + rows[i + u, pl.ds(16 * j, 16)]
                        for j in range(4)
                    )
                return acc

            zero = jnp.zeros((16,), jnp.float32)
            acc = lax.fori_loop(0, L // UNROLL, body, (zero,) * 4)
            scale = jnp.float32(1.0 / L)
            for j in range(4):
                pool_v[s, pl.ds(16 * j, 16)] = acc[j] * scale

        issue(0, rows_a, sem_a)

        def outer(k, carry):
            s = 2 * k
            issue(s + 1, rows_b, sem_b)
            wait(rows_a, sem_a)
            accum(rows_a, s)

            @pl.when(k < bpw // 2 - 1)
            def _():
                issue(s + 2, rows_a, sem_a)

            wait(rows_b, sem_b)
            accum(rows_b, s + 1)
            return carry

        lax.fori_loop(0, bpw // 2, outer, 0)

        pltpu.sync_copy(pool_v, out_hbm.at[pl.ds(base, bpw)])

    return pool_kernel(ids2, table)


def _mlp_body(x_ref, w1_ref, b1_ref, w2_ref, b2_ref, o_ref):
    h = jnp.dot(x_ref[...], w1_ref[...],
                preferred_element_type=jnp.float32) + b1_ref[...]
    h = jnp.maximum(h, 0.0)
    o_ref[...] = jnp.dot(h, w2_ref[...],
                         preferred_element_type=jnp.float32) + b2_ref[...]


@jax.jit
def kernel(text_ids, table, W1, b1, W2, b2):
    batch = text_ids.shape[0]
    ids2 = text_ids.astype(jnp.int32).reshape(NG * batch, W)
    pooled = _sc_pool(ids2, table, batch)
    return pl.pallas_call(
        _mlp_body,
        out_shape=jax.ShapeDtypeStruct((batch, D), jnp.float32),
    )(pooled, W1, b1.reshape(1, D), W2, b2.reshape(1, D))


# speedup vs baseline: 1.0844x; 1.0573x over previous
import functools

import jax
import jax.numpy as jnp
from jax import lax
from jax.experimental import pallas as pl
from jax.experimental.pallas import tpu as pltpu
from jax.experimental.pallas import tpu_sc as plsc

D = 64
L = 200
W = 40
NG = L // W
NC, NS = 2, 16
NW = NC * NS
UNROLL = 8


def _sc_pool(ids2, table, batch):
    bpw = batch // NW
    mesh = plsc.VectorSubcoreMesh(core_axis_name="c", subcore_axis_name="s")

    @functools.partial(
        pl.kernel,
        mesh=mesh,
        out_type=jax.ShapeDtypeStruct((batch, D), jnp.float32),
        compiler_params=pltpu.CompilerParams(use_tc_tiling_on_sc=False),
        scratch_types=[
            pltpu.VMEM((NG * bpw, W), jnp.int32),
            pltpu.VMEM((L, D), jnp.float32),
            pltpu.VMEM((L, D), jnp.float32),
            pltpu.VMEM((L, D), jnp.float32),
            pltpu.VMEM((L, D), jnp.float32),
            pltpu.VMEM((bpw, D), jnp.float32),
            pltpu.SemaphoreType.DMA,
            pltpu.SemaphoreType.DMA,
            pltpu.SemaphoreType.DMA,
            pltpu.SemaphoreType.DMA,
        ],
    )
    def pool_kernel(ids_hbm, table_hbm, out_hbm, ids_v, rows_a, rows_b,
                    rows_c, rows_d, pool_v, sem_a, sem_b, sem_c, sem_d):
        wid = lax.axis_index("s") * NC + lax.axis_index("c")
        base = wid * bpw

        pltpu.sync_copy(ids_hbm.at[pl.ds(NG * base, NG * bpw)], ids_v)

        def issue(s, rows, sem):
            for c in range(NG):
                pltpu.async_copy(table_hbm.at[ids_v.at[NG * s + c]],
                                 rows.at[pl.ds(W * c, W)], sem)

        def wait(rows, sem):
            pltpu.make_async_copy(table_hbm.at[pl.ds(0, L)], rows, sem).wait()

        def accum(rows, s):
            def body(k, acc):
                i = UNROLL * k
                for u in range(UNROLL):
                    acc = tuple(
                        acc[j] + rows[i + u, pl.ds(16 * j, 16)]
                        for j in range(4)
                    )
                return acc

            zero = jnp.zeros((16,), jnp.float32)
            acc = lax.fori_loop(0, L // UNROLL, body, (zero,) * 4)
            scale = jnp.float32(1.0 / L)
            for j in range(4):
                pool_v[s, pl.ds(16 * j, 16)] = acc[j] * scale

        bufs = ((rows_a, sem_a), (rows_b, sem_b), (rows_c, sem_c),
                (rows_d, sem_d))
        NBUF = len(bufs)

        for b in range(NBUF - 1):
            issue(b, *bufs[b])

        def outer(k, carry):
            s = NBUF * k
            for b in range(NBUF):
                rows, sem = bufs[b]
                nxt = s + b + NBUF - 1

                @pl.when(nxt < bpw)
                def _(nxt=nxt, nb=bufs[(b + NBUF - 1) % NBUF]):
                    issue(nxt, *nb)

                wait(rows, sem)
                accum(rows, s + b)
            return carry

        lax.fori_loop(0, bpw // NBUF, outer, 0)

        pltpu.sync_copy(pool_v, out_hbm.at[pl.ds(base, bpw)])

    return pool_kernel(ids2, table)


def _mlp_body(x_ref, w1_ref, b1_ref, w2_ref, b2_ref, o_ref):
    h = jnp.dot(x_ref[...], w1_ref[...],
                preferred_element_type=jnp.float32) + b1_ref[...]
    h = jnp.maximum(h, 0.0)
    o_ref[...] = jnp.dot(h, w2_ref[...],
                         preferred_element_type=jnp.float32) + b2_ref[...]


@jax.jit
def kernel(text_ids, table, W1, b1, W2, b2):
    batch = text_ids.shape[0]
    ids2 = text_ids.astype(jnp.int32).reshape(NG * batch, W)
    pooled = _sc_pool(ids2, table, batch)
    return pl.pallas_call(
        _mlp_body,
        out_shape=jax.ShapeDtypeStruct((batch, D), jnp.float32),
    )(pooled, W1, b1.reshape(1, D), W2, b2.reshape(1, D))
